# baseline (speedup 1.0000x reference)
import jax
import jax.numpy as jnp
from jax.experimental import pallas as pl
from jax.experimental.pallas import tpu as pltpu


def _tc_body(votes_ref, w_ref, out_ref):
    w = w_ref[...]
    total = jnp.sum(w)
    counts = jnp.sum(w * votes_ref[...].astype(jnp.float32), axis=0)
    out_ref[...] = jnp.where(counts + counts > total, 1, 0).astype(jnp.int32)


def kernel(votes, vote_weights):
    V, B = votes.shape
    NB = 131072
    grid = (B // NB,)
    w2 = vote_weights.astype(jnp.float32).reshape(V, 1)
    return pl.pallas_call(
        _tc_body,
        grid=grid,
        in_specs=[
            pl.BlockSpec((V, NB), lambda i: (0, i)),
            pl.BlockSpec((V, 1), lambda i: (0, 0)),
        ],
        out_specs=pl.BlockSpec((NB,), lambda i: (i,)),
        out_shape=jax.ShapeDtypeStruct((B,), jnp.int32),
    )(votes, w2)
